# Initial kernel scaffold; baseline (speedup 1.0000x reference)
import functools

import jax
import jax.numpy as jnp
from jax import lax
from jax.experimental import pallas as pl
from jax.experimental.pallas import tpu as pltpu
from jax.experimental.pallas import tpu_sc as plsc

D_MODEL = 128
NUM_WORKERS = 32
CHUNK = 256
GATHER_W = 128


def _body(x_hbm, idx_hbm, pe_hbm, out_hbm, xbuf, idxbuf, perows, sem):
    n_rows = x_hbm.shape[0]
    rows_per_worker = n_rows // NUM_WORKERS
    n_chunks = rows_per_worker // CHUNK
    wid = lax.axis_index("s") * 2 + lax.axis_index("c")
    base = wid * rows_per_worker

    def chunk_body(g, carry):
        row0 = base + g * CHUNK
        pltpu.sync_copy(x_hbm.at[pl.ds(row0, CHUNK), :], xbuf)
        pltpu.sync_copy(idx_hbm.at[pl.ds(row0 // GATHER_W, CHUNK // GATHER_W), :],
                        idxbuf)
        for j in range(CHUNK // GATHER_W):
            pltpu.async_copy(
                pe_hbm.at[idxbuf.at[j]],
                perows.at[pl.ds(j * GATHER_W, GATHER_W), :],
                sem,
            ).wait()

        def add_row(r, inner):
            for c in range(D_MODEL // 16):
                v = perows[r, pl.ds(c * 16, 16)]
                plsc.addupdate(xbuf.at[r, pl.ds(c * 16, 16)], v)
            return inner

        lax.fori_loop(0, CHUNK, add_row, 0)
        pltpu.sync_copy(xbuf, out_hbm.at[pl.ds(row0, CHUNK), :])
        return carry

    lax.fori_loop(0, n_chunks, chunk_body, 0)


def kernel(x, frame_indices, pe):
    b, l, d = x.shape
    n = b * l
    x2 = x.reshape(n, d)
    idx2 = frame_indices.astype(jnp.int32).reshape(n // GATHER_W, GATHER_W)

    mesh = plsc.VectorSubcoreMesh(core_axis_name="c", subcore_axis_name="s")
    run = pl.kernel(
        _body,
        out_type=jax.ShapeDtypeStruct((n, d), jnp.float32),
        mesh=mesh,
        scratch_types=[
            pltpu.VMEM((CHUNK, D_MODEL), jnp.float32),
            pltpu.VMEM((CHUNK // GATHER_W, GATHER_W), jnp.int32),
            pltpu.VMEM((CHUNK, D_MODEL), jnp.float32),
            pltpu.SemaphoreType.DMA,
        ],
    )
    out = run(x2, idx2, pe)
    return out.reshape(b, l, d)


# speedup vs baseline: 3.3806x; 3.3806x over previous
import jax
import jax.numpy as jnp
from jax import lax
from jax.experimental import pallas as pl
from jax.experimental.pallas import tpu as pltpu
from jax.experimental.pallas import tpu_sc as plsc

D_MODEL = 128
NUM_WORKERS = 32
CHUNK = 256
GATHER_W = 128


def _body(x_hbm, idx_hbm, pe_hbm, out_hbm, xbuf, idxbuf, perows, sem):
    n_rows = x_hbm.shape[0]
    rows_per_worker = n_rows // NUM_WORKERS
    n_chunks = rows_per_worker // CHUNK
    idx_rows_per_worker = rows_per_worker // GATHER_W
    wid = lax.axis_index("s") * 2 + lax.axis_index("c")
    base = wid * rows_per_worker

    pltpu.sync_copy(idx_hbm.at[pl.ds(wid * idx_rows_per_worker,
                                     idx_rows_per_worker), :], idxbuf)

    def chunk_body(g, carry):
        row0 = base + g * CHUNK
        pltpu.sync_copy(x_hbm.at[pl.ds(row0, CHUNK), :], xbuf)
        for j in range(CHUNK // GATHER_W):
            pltpu.async_copy(
                pe_hbm.at[idxbuf.at[g * (CHUNK // GATHER_W) + j]],
                perows.at[pl.ds(j * GATHER_W, GATHER_W), :],
                sem,
            ).wait()

        def add_row(r, inner):
            for c in range(D_MODEL // 16):
                v = perows[r, pl.ds(c * 16, 16)]
                plsc.addupdate(xbuf.at[r, pl.ds(c * 16, 16)], v)
            return inner

        lax.fori_loop(0, CHUNK, add_row, 0)
        pltpu.sync_copy(xbuf, out_hbm.at[pl.ds(row0, CHUNK), :])
        return carry

    lax.fori_loop(0, n_chunks, chunk_body, 0)


def kernel(x, frame_indices, pe):
    b, l, d = x.shape
    n = b * l
    x2 = x.reshape(n, d)
    idx2 = frame_indices.astype(jnp.int32).reshape(n // GATHER_W, GATHER_W)
    idx_rows_per_worker = (n // NUM_WORKERS) // GATHER_W

    mesh = plsc.VectorSubcoreMesh(core_axis_name="c", subcore_axis_name="s")
    run = pl.kernel(
        _body,
        out_type=jax.ShapeDtypeStruct((n, d), jnp.float32),
        mesh=mesh,
        scratch_types=[
            pltpu.VMEM((CHUNK, D_MODEL), jnp.float32),
            pltpu.VMEM((idx_rows_per_worker, GATHER_W), jnp.int32),
            pltpu.VMEM((CHUNK, D_MODEL), jnp.float32),
            pltpu.SemaphoreType.DMA,
        ],
    )
    out = run(x2, idx2, pe)
    return out.reshape(b, l, d)


# speedup vs baseline: 4.3166x; 1.2769x over previous
import jax
import jax.numpy as jnp
from jax import lax
from jax.experimental import pallas as pl
from jax.experimental.pallas import tpu as pltpu
from jax.experimental.pallas import tpu_sc as plsc

D_MODEL = 128
NUM_WORKERS = 32
CHUNK = 128
DEPTH = 3


def _body(x_hbm, idx_hbm, pe_hbm, out_hbm,
          xb0, xb1, xb2, pr0, pr1, pr2, idxbuf,
          seml, semg, sems):
    xbufs = (xb0, xb1, xb2)
    prows = (pr0, pr1, pr2)
    n_rows = x_hbm.shape[0]
    rows_per_worker = n_rows // NUM_WORKERS
    n_chunks = rows_per_worker // CHUNK
    wid = lax.axis_index("s") * 2 + lax.axis_index("c")
    base = wid * rows_per_worker

    def start_load(g, s):
        pltpu.async_copy(x_hbm.at[pl.ds(base + g * CHUNK, CHUNK), :],
                         xbufs[s], seml.at[s])

    def wait_load(g, s):
        pltpu.make_async_copy(x_hbm.at[pl.ds(base + g * CHUNK, CHUNK), :],
                              xbufs[s], seml.at[s]).wait()

    def start_gather(g, s):
        pltpu.async_copy(pe_hbm.at[idxbuf.at[g]], prows[s], semg.at[s])

    def wait_gather(g, s):
        pltpu.make_async_copy(pe_hbm.at[idxbuf.at[g]], prows[s],
                              semg.at[s]).wait()

    def start_store(g, s):
        pltpu.async_copy(prows[s], out_hbm.at[pl.ds(base + g * CHUNK, CHUNK), :],
                         sems.at[s])

    def wait_store(g, s):
        pltpu.make_async_copy(prows[s],
                              out_hbm.at[pl.ds(base + g * CHUNK, CHUNK), :],
                              sems.at[s]).wait()

    pltpu.sync_copy(idx_hbm.at[pl.ds(wid * n_chunks, n_chunks), :], idxbuf)

    for s in range(DEPTH):
        start_load(s, s)
    start_gather(0, 0)

    n_steps = -(-n_chunks // DEPTH)

    def step(i, carry):
        for s in range(DEPTH):
            g = i * DEPTH + s

            @pl.when(g < n_chunks)
            def _():
                wait_load(g, s)
                wait_gather(g, s)

                def add_row(r, inner):
                    for c in range(D_MODEL // 16):
                        v = xbufs[s][r, pl.ds(c * 16, 16)]
                        plsc.addupdate(prows[s].at[r, pl.ds(c * 16, 16)], v)
                    return inner

                lax.fori_loop(0, CHUNK, add_row, 0)
                start_store(g, s)

                @pl.when(g + DEPTH < n_chunks)
                def _():
                    start_load(g + DEPTH, s)

                s1 = (s + 1) % DEPTH

                @pl.when(jnp.logical_and(g >= DEPTH - 1,
                                         g + 1 < n_chunks))
                def _():
                    wait_store(g + 1 - DEPTH, s1)

                @pl.when(g + 1 < n_chunks)
                def _():
                    start_gather(g + 1, s1)

        return carry

    lax.fori_loop(0, n_steps, step, 0)

    for k in range(DEPTH):
        g = n_chunks - DEPTH + k
        wait_store(g, g % DEPTH)


def kernel(x, frame_indices, pe):
    b, l, d = x.shape
    n = b * l
    x2 = x.reshape(n, d)
    idx2 = frame_indices.astype(jnp.int32).reshape(n // CHUNK, CHUNK)
    n_chunks = (n // NUM_WORKERS) // CHUNK

    mesh = plsc.VectorSubcoreMesh(core_axis_name="c", subcore_axis_name="s")
    run = pl.kernel(
        _body,
        out_type=jax.ShapeDtypeStruct((n, d), jnp.float32),
        mesh=mesh,
        scratch_types=(
            [pltpu.VMEM((CHUNK, D_MODEL), jnp.float32) for _ in range(DEPTH)]
            + [pltpu.VMEM((CHUNK, D_MODEL), jnp.float32) for _ in range(DEPTH)]
            + [pltpu.VMEM((n_chunks, CHUNK), jnp.int32)]
            + [pltpu.SemaphoreType.DMA((DEPTH,)),
               pltpu.SemaphoreType.DMA((DEPTH,)),
               pltpu.SemaphoreType.DMA((DEPTH,))]
        ),
    )
    out = run(x2, idx2, pe)
    return out.reshape(b, l, d)
